# Initial kernel scaffold; baseline (speedup 1.0000x reference)
import jax
import jax.numpy as jnp
from jax.experimental import pallas as pl


def kernel(position_ids, maps):
    raise NotImplementedError("write your pallas kernel here")


# speedup vs baseline: 2.1629x; 2.1629x over previous
import functools

import jax
import jax.numpy as jnp
from jax import lax
from jax.experimental import pallas as pl
from jax.experimental.pallas import tpu as pltpu
from jax.experimental.pallas import tpu_sc as plsc

DIM = 32
NUM_HEADS = 16
ROW = NUM_HEADS * DIM * DIM

NC = 2
NS = 16
NW = NC * NS

CH = 4


def _gather_body(idx_hbm, table_hbm, out_hbm, idx_v, rows_v, sem):
    wid = lax.axis_index("s") * NC + lax.axis_index("c")
    nchunks = idx_v.shape[0]
    row_base = wid * (nchunks * CH)
    pltpu.sync_copy(idx_hbm.at[wid], idx_v)

    def step(i, carry):
        pltpu.async_copy(table_hbm.at[idx_v.at[i]], rows_v, sem).wait()
        pltpu.sync_copy(rows_v, out_hbm.at[pl.ds(row_base + i * CH, CH)])
        return carry

    lax.fori_loop(0, nchunks, step, 0)


@functools.partial(jax.jit, static_argnums=(2,))
def _sc_gather(idx3, table2, total_rows):
    nchunks = total_rows // (NW * CH)
    mesh = plsc.VectorSubcoreMesh(core_axis_name="c", subcore_axis_name="s")
    return pl.kernel(
        _gather_body,
        out_type=jax.ShapeDtypeStruct((total_rows, ROW), jnp.float32),
        mesh=mesh,
        scratch_types=[
            pltpu.VMEM((nchunks, CH), jnp.int32),
            pltpu.VMEM((CH, ROW), jnp.float32),
            pltpu.SemaphoreType.DMA,
        ],
    )(idx3, table2)


def kernel(position_ids, maps):
    batch, size = position_ids.shape
    total = batch * size
    table2 = maps.reshape(maps.shape[0], ROW)
    idx3 = position_ids.reshape(NW, total // (NW * CH), CH).astype(jnp.int32)
    out = _sc_gather(idx3, table2, total)
    return out.reshape(batch, size, NUM_HEADS, DIM, DIM)


# speedup vs baseline: 2.1757x; 1.0059x over previous
import functools

import jax
import jax.numpy as jnp
from jax import lax
from jax.experimental import pallas as pl
from jax.experimental.pallas import tpu as pltpu
from jax.experimental.pallas import tpu_sc as plsc

DIM = 32
NUM_HEADS = 16
ROW = NUM_HEADS * DIM * DIM

NC = 2
NS = 16
NW = NC * NS

CH = 2


def _gather_body(idx_hbm, table_hbm, out_hbm, idx_v, buf0, buf1,
                 gs0, gs1, ss0, ss1):
    wid = lax.axis_index("s") * NC + lax.axis_index("c")
    nch = idx_v.shape[0]
    row_base = wid * (nch * CH)
    pltpu.sync_copy(idx_hbm.at[wid], idx_v)

    def g_start(c, buf, sem):
        pltpu.async_copy(table_hbm.at[idx_v.at[c]], buf, sem)

    def g_wait(c, buf, sem):
        pltpu.make_async_copy(table_hbm.at[idx_v.at[c]], buf, sem).wait()

    def s_start(c, buf, sem):
        pltpu.async_copy(buf, out_hbm.at[pl.ds(row_base + c * CH, CH)], sem)

    def s_wait(c, buf, sem):
        pltpu.make_async_copy(
            buf, out_hbm.at[pl.ds(row_base + c * CH, CH)], sem).wait()

    g_start(0, buf0, gs0)
    g_start(1, buf1, gs1)
    T = nch // 2

    def body(t, carry):
        c0 = 2 * t
        c1 = c0 + 1
        g_wait(c0, buf0, gs0)
        s_start(c0, buf0, ss0)
        g_wait(c1, buf1, gs1)
        s_start(c1, buf1, ss1)

        @pl.when(t < T - 1)
        def _():
            s_wait(c0, buf0, ss0)
            g_start(c0 + 2, buf0, gs0)
            s_wait(c1, buf1, ss1)
            g_start(c1 + 2, buf1, gs1)

        return carry

    lax.fori_loop(0, T, body, 0)
    s_wait(nch - 2, buf0, ss0)
    s_wait(nch - 1, buf1, ss1)


@functools.partial(jax.jit, static_argnums=(2,))
def _sc_gather(idx3, table2, total_rows):
    nchunks = total_rows // (NW * CH)
    mesh = plsc.VectorSubcoreMesh(core_axis_name="c", subcore_axis_name="s")
    return pl.kernel(
        _gather_body,
        out_type=jax.ShapeDtypeStruct((total_rows, ROW), jnp.float32),
        mesh=mesh,
        scratch_types=[
            pltpu.VMEM((nchunks, CH), jnp.int32),
            pltpu.VMEM((CH, ROW), jnp.float32),
            pltpu.VMEM((CH, ROW), jnp.float32),
            pltpu.SemaphoreType.DMA,
            pltpu.SemaphoreType.DMA,
            pltpu.SemaphoreType.DMA,
            pltpu.SemaphoreType.DMA,
        ],
    )(idx3, table2)


def kernel(position_ids, maps):
    batch, size = position_ids.shape
    total = batch * size
    table2 = maps.reshape(maps.shape[0], ROW)
    idx3 = position_ids.reshape(NW, total // (NW * CH), CH).astype(jnp.int32)
    out = _sc_gather(idx3, table2, total)
    return out.reshape(batch, size, NUM_HEADS, DIM, DIM)
